# baseline (speedup 1.0000x reference)
import functools

import jax
import jax.numpy as jnp
from jax import lax
from jax.experimental import pallas as pl
from jax.experimental.pallas import tpu as pltpu
from jax.experimental.pallas import tpu_sc as plsc

E = 8
M = 2048
K = 768
N = 768
BM = 256
BM_LOG2 = 8
MP = M + E * BM
NB = MP // BM

NC = 2
NS = 16
NW = NC * NS
TPW = M // NW
CPW = TPW // 16
NCHUNK = M // 16

_MESH = plsc.VectorSubcoreMesh(core_axis_name="c", subcore_axis_name="s")


def _count_chunk(ids_v, c, cnts):
    chunk = ids_v[pl.ds(c * 16, 16)]
    return tuple(
        cnts[e] + jnp.where(chunk == e, 1, 0).astype(jnp.int32)
        for e in range(E))


def _prefix16(v, stage, lane):
    for s in (1, 2, 4, 8):
        stage[...] = v
        g = plsc.load_gather(stage, [jnp.maximum(lane - s, 0)])
        v = v + jnp.where(lane >= s, g, 0)
    return v


def _splat15(v, stage, lane):
    stage[...] = v
    return plsc.load_gather(stage, [(lane * 0) + 15])


@functools.partial(
    pl.kernel,
    mesh=_MESH,
    out_type=(
        jax.ShapeDtypeStruct((MP, K), jnp.float32),
        jax.ShapeDtypeStruct((M,), jnp.int32),
        jax.ShapeDtypeStruct((64,), jnp.int32),
    ),
    scratch_types=[
        pltpu.VMEM((M,), jnp.int32),
        pltpu.VMEM((TPW,), jnp.int32),
        pltpu.VMEM((TPW, K), jnp.float32),
        pltpu.VMEM((64,), jnp.int32),
        pltpu.VMEM((16,), jnp.int32),
        pltpu.SemaphoreType.DMA,
    ],
    compiler_params=pltpu.CompilerParams(needs_layout_passes=False),
)
def _sc_prepare(hid_hbm, ids_hbm, xpad_hbm, pos_hbm, bmap_hbm,
                ids_v, pos_v, rows_v, bmap_v, stage_v, sem):
    wid = lax.axis_index("s") * NC + lax.axis_index("c")
    base = wid * TPW
    lane = lax.iota(jnp.int32, 16)
    pltpu.sync_copy(ids_hbm, ids_v)

    zero = jnp.zeros((16,), jnp.int32)
    acc8 = lax.fori_loop(0, wid * CPW,
                         lambda c, a: _count_chunk(ids_v, c, a),
                         tuple(zero for _ in range(E)))
    cnts = [_splat15(_prefix16(acc8[e], stage_v, lane), stage_v, lane)
            for e in range(E)]
    ranks = []
    own = []
    for j in range(CPW):
        chunk = ids_v[pl.ds(base + j * 16, 16)]
        own.append(chunk)
        r = zero
        for e in range(E):
            m = chunk == e
            incl = _prefix16(jnp.where(m, 1, 0).astype(jnp.int32),
                             stage_v, lane)
            r = jnp.where(m, cnts[e] + incl - 1, r)
            cnts[e] = cnts[e] + _splat15(incl, stage_v, lane)
        ranks.append(r)
    rest8 = lax.fori_loop((wid + 1) * CPW, NCHUNK,
                          lambda c, a: _count_chunk(ids_v, c, a),
                          tuple(zero for _ in range(E)))
    for e in range(E):
        cnts[e] = cnts[e] + _splat15(
            _prefix16(rest8[e], stage_v, lane), stage_v, lane)
    acc = zero
    starts = []
    cumb = []
    for e in range(E):
        starts.append(acc)
        nblk = (cnts[e] + (BM - 1)) >> BM_LOG2
        acc = acc + (nblk << BM_LOG2)
        cumb.append(acc >> BM_LOG2)
    for j in range(CPW):
        p = ranks[j]
        for e in range(E):
            p = jnp.where(own[j] == e, p + starts[e], p)
        pos_v[pl.ds(j * 16, 16)] = p
    pltpu.sync_copy(pos_v, pos_hbm.at[pl.ds(base, TPW)])
    pltpu.sync_copy(hid_hbm.at[pl.ds(base, TPW)], rows_v)
    pltpu.async_copy(rows_v, xpad_hbm.at[pos_v], sem).wait()

    @pl.when(wid == 0)
    def _():
        lane = lax.iota(jnp.int32, 16)
        for c in range(2):
            bl = lane + c * 16
            be = zero
            for e in range(E):
                be = be + jnp.where(bl >= cumb[e], 1, 0)
            bmap_v[pl.ds(c * 16, 16)] = jnp.minimum(be, E - 1)
            bmap_v[pl.ds(32 + c * 16, 16)] = jnp.where(bl < cumb[E - 1], 1, 0)
        pltpu.sync_copy(bmap_v, bmap_hbm)


@functools.partial(
    pl.kernel,
    mesh=_MESH,
    out_type=jax.ShapeDtypeStruct((M, K), jnp.float32),
    scratch_types=[
        pltpu.VMEM((TPW,), jnp.int32),
        pltpu.VMEM((TPW, K), jnp.float32),
        pltpu.VMEM((TPW,), jnp.float32),
        pltpu.SemaphoreType.DMA,
    ],
    compiler_params=pltpu.CompilerParams(needs_layout_passes=False),
)
def _sc_finalize(outpad_hbm, pos_hbm, wts_hbm, out_hbm,
                 idx_v, rows_v, w_v, sem):
    wid = lax.axis_index("s") * NC + lax.axis_index("c")
    base = wid * TPW
    pltpu.sync_copy(pos_hbm.at[pl.ds(base, TPW)], idx_v)
    pltpu.async_copy(outpad_hbm.at[idx_v], rows_v, sem).wait()
    pltpu.sync_copy(wts_hbm.at[pl.ds(base, TPW)], w_v)

    def row_body(r, carry):
        ws = plsc.load_gather(w_v, [jnp.zeros((16,), jnp.int32) + r])
        for c in range(K // 16):
            rows_v[r, pl.ds(c * 16, 16)] = rows_v[r, pl.ds(c * 16, 16)] * ws
        return carry

    lax.fori_loop(0, TPW, row_body, 0)
    pltpu.sync_copy(rows_v, out_hbm.at[pl.ds(base, TPW)])


def _moe_gemm_body(bmap_ref, x_ref, w1_ref, w2_ref, out_ref):
    @pl.when(bmap_ref[32 + pl.program_id(0)] == 1)
    def _():
        x = x_ref[...]
        w1e = w1_ref[0]
        h = lax.dot_general(x, w1e, (((1,), (1,)), ((), ())),
                            preferred_element_type=jnp.float32)
        gate = h[:, :N]
        up = h[:, N:]
        act = gate * jax.nn.sigmoid(gate) * up
        w2e = w2_ref[0]
        out_ref[...] = lax.dot_general(act, w2e, (((1,), (1,)), ((), ())),
                                       preferred_element_type=jnp.float32)


def _grouped_gemm(x_pad, w1, w2, bmap):
    grid_spec = pltpu.PrefetchScalarGridSpec(
        num_scalar_prefetch=1,
        grid=(NB,),
        in_specs=[
            pl.BlockSpec((BM, K), lambda b, em: (b, 0)),
            pl.BlockSpec((1, 2 * N, K), lambda b, em: (em[b], 0, 0)),
            pl.BlockSpec((1, K, N), lambda b, em: (em[b], 0, 0)),
        ],
        out_specs=pl.BlockSpec((BM, K), lambda b, em: (b, 0)),
    )
    return pl.pallas_call(
        _moe_gemm_body,
        grid_spec=grid_spec,
        out_shape=jax.ShapeDtypeStruct((MP, K), jnp.float32),
    )(bmap, x_pad, w1, w2)


def kernel(hidden_states, w1, w2, topk_weights, topk_ids):
    ids = topk_ids.reshape(M).astype(jnp.int32)
    wts = topk_weights.reshape(M).astype(jnp.float32)
    x_pad, pos, bmap = _sc_prepare(hidden_states, ids)
    out_pad = _grouped_gemm(x_pad, w1, w2, bmap)
    return _sc_finalize(out_pad, pos, wts)


# speedup vs baseline: 1.0726x; 1.0726x over previous
import functools

import jax
import jax.numpy as jnp
from jax import lax
from jax.experimental import pallas as pl
from jax.experimental.pallas import tpu as pltpu
from jax.experimental.pallas import tpu_sc as plsc

E = 8
M = 2048
K = 768
N = 768
BM = 256
BM_LOG2 = 8
MP = M + E * BM
NB = MP // BM

NC = 2
NS = 16
NW = NC * NS
TPW = M // NW
CPW = TPW // 16
NCHUNK = M // 16

_MESH = plsc.VectorSubcoreMesh(core_axis_name="c", subcore_axis_name="s")


def _count_chunk(ids_v, c, cnts):
    chunk = ids_v[pl.ds(c * 16, 16)]
    return tuple(
        cnts[e] + jnp.where(chunk == e, 1, 0).astype(jnp.int32)
        for e in range(E))


def _prefix16(v, stage, lane):
    for s in (1, 2, 4, 8):
        stage[...] = v
        g = plsc.load_gather(stage, [jnp.maximum(lane - s, 0)])
        v = v + jnp.where(lane >= s, g, 0)
    return v


def _splat15(v, stage, lane):
    stage[...] = v
    return plsc.load_gather(stage, [(lane * 0) + 15])


@functools.partial(
    pl.kernel,
    mesh=_MESH,
    out_type=(
        jax.ShapeDtypeStruct((MP, K), jnp.float32),
        jax.ShapeDtypeStruct((M,), jnp.int32),
        jax.ShapeDtypeStruct((64,), jnp.int32),
    ),
    scratch_types=[
        pltpu.VMEM((M,), jnp.int32),
        pltpu.VMEM((TPW,), jnp.int32),
        pltpu.VMEM((TPW, K), jnp.float32),
        pltpu.VMEM((64,), jnp.int32),
        pltpu.VMEM((16,), jnp.int32),
        pltpu.SemaphoreType.DMA,
    ],
    compiler_params=pltpu.CompilerParams(needs_layout_passes=False),
)
def _sc_prepare(hid_hbm, ids_hbm, xpad_hbm, pos_hbm, bmap_hbm,
                ids_v, pos_v, rows_v, bmap_v, stage_v, sem):
    wid = lax.axis_index("s") * NC + lax.axis_index("c")
    base = wid * TPW
    lane = lax.iota(jnp.int32, 16)
    rows_dma = pltpu.async_copy(hid_hbm.at[pl.ds(base, TPW)], rows_v, sem)
    pltpu.sync_copy(ids_hbm, ids_v)

    zero = jnp.zeros((16,), jnp.int32)
    acc8 = lax.fori_loop(0, wid * CPW,
                         lambda c, a: _count_chunk(ids_v, c, a),
                         tuple(zero for _ in range(E)))
    cnts = [_splat15(_prefix16(acc8[e], stage_v, lane), stage_v, lane)
            for e in range(E)]
    ranks = []
    own = []
    for j in range(CPW):
        chunk = ids_v[pl.ds(base + j * 16, 16)]
        own.append(chunk)
        r = zero
        for e in range(E):
            m = chunk == e
            incl = _prefix16(jnp.where(m, 1, 0).astype(jnp.int32),
                             stage_v, lane)
            r = jnp.where(m, cnts[e] + incl - 1, r)
            cnts[e] = cnts[e] + _splat15(incl, stage_v, lane)
        ranks.append(r)
    rest8 = lax.fori_loop((wid + 1) * CPW, NCHUNK,
                          lambda c, a: _count_chunk(ids_v, c, a),
                          tuple(zero for _ in range(E)))
    for e in range(E):
        cnts[e] = cnts[e] + _splat15(
            _prefix16(rest8[e], stage_v, lane), stage_v, lane)
    acc = zero
    starts = []
    cumb = []
    for e in range(E):
        starts.append(acc)
        nblk = (cnts[e] + (BM - 1)) >> BM_LOG2
        acc = acc + (nblk << BM_LOG2)
        cumb.append(acc >> BM_LOG2)
    for j in range(CPW):
        p = ranks[j]
        for e in range(E):
            p = jnp.where(own[j] == e, p + starts[e], p)
        pos_v[pl.ds(j * 16, 16)] = p
    pltpu.sync_copy(pos_v, pos_hbm.at[pl.ds(base, TPW)])
    rows_dma.wait()
    pltpu.async_copy(rows_v, xpad_hbm.at[pos_v], sem).wait()

    @pl.when(wid == 0)
    def _():
        lane = lax.iota(jnp.int32, 16)
        for c in range(2):
            bl = lane + c * 16
            be = zero
            for e in range(E):
                be = be + jnp.where(bl >= cumb[e], 1, 0)
            bmap_v[pl.ds(c * 16, 16)] = jnp.minimum(be, E - 1)
            bmap_v[pl.ds(32 + c * 16, 16)] = jnp.where(bl < cumb[E - 1], 1, 0)
        pltpu.sync_copy(bmap_v, bmap_hbm)


@functools.partial(
    pl.kernel,
    mesh=_MESH,
    out_type=jax.ShapeDtypeStruct((M, K), jnp.float32),
    scratch_types=[
        pltpu.VMEM((TPW,), jnp.int32),
        pltpu.VMEM((TPW, K), jnp.float32),
        pltpu.VMEM((TPW,), jnp.float32),
        pltpu.SemaphoreType.DMA,
    ],
    compiler_params=pltpu.CompilerParams(needs_layout_passes=False),
)
def _sc_finalize(outpad_hbm, pos_hbm, wts_hbm, out_hbm,
                 idx_v, rows_v, w_v, sem):
    wid = lax.axis_index("s") * NC + lax.axis_index("c")
    base = wid * TPW
    pltpu.sync_copy(pos_hbm.at[pl.ds(base, TPW)], idx_v)
    pltpu.async_copy(outpad_hbm.at[idx_v], rows_v, sem).wait()
    pltpu.sync_copy(wts_hbm.at[pl.ds(base, TPW)], w_v)

    def row_body(r, carry):
        ws = plsc.load_gather(w_v, [jnp.zeros((16,), jnp.int32) + r])
        for c in range(K // 16):
            rows_v[r, pl.ds(c * 16, 16)] = rows_v[r, pl.ds(c * 16, 16)] * ws
        return carry

    lax.fori_loop(0, TPW, row_body, 0)
    pltpu.sync_copy(rows_v, out_hbm.at[pl.ds(base, TPW)])


def _moe_gemm_body(bmap_ref, x_ref, w1_ref, w2_ref, out_ref):
    @pl.when(bmap_ref[32 + pl.program_id(0)] == 1)
    def _():
        x = x_ref[...]
        w1e = w1_ref[0]
        h = lax.dot_general(x, w1e, (((1,), (1,)), ((), ())),
                            preferred_element_type=jnp.float32)
        gate = h[:, :N]
        up = h[:, N:]
        act = gate * jax.nn.sigmoid(gate) * up
        w2e = w2_ref[0]
        out_ref[...] = lax.dot_general(act, w2e, (((1,), (1,)), ((), ())),
                                       preferred_element_type=jnp.float32)


def _grouped_gemm(x_pad, w1, w2, bmap):
    grid_spec = pltpu.PrefetchScalarGridSpec(
        num_scalar_prefetch=1,
        grid=(NB,),
        in_specs=[
            pl.BlockSpec((BM, K),
                         lambda b, em: (jnp.where(em[32 + b] == 1, b, 0), 0)),
            pl.BlockSpec((1, 2 * N, K), lambda b, em: (em[b], 0, 0)),
            pl.BlockSpec((1, K, N), lambda b, em: (em[b], 0, 0)),
        ],
        out_specs=pl.BlockSpec(
            (BM, K), lambda b, em: (jnp.where(em[32 + b] == 1, b, NB - 1), 0)),
    )
    return pl.pallas_call(
        _moe_gemm_body,
        grid_spec=grid_spec,
        out_shape=jax.ShapeDtypeStruct((MP, K), jnp.float32),
    )(bmap, x_pad, w1, w2)


def kernel(hidden_states, w1, w2, topk_weights, topk_ids):
    ids = topk_ids.reshape(M).astype(jnp.int32)
    wts = topk_weights.reshape(M).astype(jnp.float32)
    x_pad, pos, bmap = _sc_prepare(hidden_states, ids)
    out_pad = _grouped_gemm(x_pad, w1, w2, bmap)
    return _sc_finalize(out_pad, pos, wts)
